# baseline (speedup 1.0000x reference)
import functools

import jax
import jax.numpy as jnp
from jax.experimental import pallas as pl
from jax.experimental.pallas import tpu as pltpu


def _bisect_steps(a, k, lo, hi, steps):
    for _ in range(steps):
        mid = jnp.floor((lo + hi + 1.0) * 0.5)
        cnt = jnp.sum((a >= mid).astype(jnp.float32), axis=-1, keepdims=True)
        ge = cnt >= k
        lo = jnp.where(ge, mid, lo)
        hi = jnp.where(ge, hi, mid - 1.0)
    return lo, hi


def _bisect_finish(a, k, lo, hi):
    def cond(carry):
        lo, hi = carry
        return jnp.max(hi - lo) > 0.0

    def body(carry):
        return _bisect_steps(a, k, *carry, steps=2)

    lo, _ = jax.lax.while_loop(cond, body, (lo, hi))
    return lo


def _row_min_max(a):
    n = a.shape[-1]
    mn = a[:, :128]
    mx = mn
    for c in range(128, n, 128):
        blk = a[:, c : c + 128]
        mn = jnp.minimum(mn, blk)
        mx = jnp.maximum(mx, blk)
    return (
        jnp.min(mn, axis=-1, keepdims=True),
        jnp.max(mx, axis=-1, keepdims=True),
    )


def _wta_block(x_ref, wxh_ref, wcat_ref, y_ref, cat_ref, ay_ref, nblk):
    i = pl.program_id(0)

    @pl.when(i == 0)
    def _init():
        ay_ref[1] = jnp.zeros_like(ay_ref[1])

    x = x_ref[...].astype(jnp.bfloat16)
    a_h = jnp.dot(x, wxh_ref[...], preferred_element_type=jnp.float32)
    hi_h = jnp.max(a_h, axis=-1, keepdims=True)
    lo_h, hi_h = _bisect_steps(a_h, 13, jnp.zeros_like(hi_h), hi_h, steps=4)
    thr_h = _bisect_finish(a_h, 13, lo_h, hi_h)
    h = jnp.where(a_h >= thr_h, a_h, 0.0).astype(jnp.bfloat16)
    cat_ref[:, : x.shape[1]] = x
    cat_ref[:, x.shape[1] :] = h

    parity = jax.lax.rem(i, 2)
    a_y_prev = ay_ref[1 - parity]
    lo_y, hi_y = _row_min_max(a_y_prev)
    lo_y, hi_y = _bisect_steps(a_y_prev, 51, lo_y, hi_y, steps=6)
    a_y = jnp.dot(cat_ref[...], wcat_ref[...], preferred_element_type=jnp.float32)
    ay_ref[parity] = a_y

    thr_y = _bisect_finish(a_y_prev, 51, lo_y, hi_y)
    y_ref[...] = jnp.where(a_y_prev >= thr_y, a_y_prev, 0.0)


@jax.jit
def _wta(x, w_xh, w_cat):
    B, NX = x.shape
    NH = w_xh.shape[1]
    NY = w_cat.shape[1]
    BLK = 1024
    nblk = B // BLK
    return pl.pallas_call(
        functools.partial(_wta_block, nblk=nblk),
        grid=(nblk + 1,),
        in_specs=[
            pl.BlockSpec((BLK, NX), lambda i: (jnp.minimum(i, nblk - 1), 0)),
            pl.BlockSpec((NX, NH), lambda i: (0, 0)),
            pl.BlockSpec((NX + NH, NY), lambda i: (0, 0)),
        ],
        out_specs=pl.BlockSpec((BLK, NY), lambda i: (jnp.maximum(i - 1, 0), 0)),
        out_shape=jax.ShapeDtypeStruct((B, NY), jnp.float32),
        scratch_shapes=[
            pltpu.VMEM((BLK, NX + NH), jnp.bfloat16),
            pltpu.VMEM((2, BLK, NY), jnp.float32),
        ],
    )(x, w_xh, w_cat)


def kernel(x, w_xy, w_xh, w_hy, k_y, k_h):
    w_cat = jnp.concatenate(
        [w_xy.astype(jnp.bfloat16), -w_hy.astype(jnp.bfloat16)], axis=0
    )
    return _wta(x, w_xh.astype(jnp.bfloat16), w_cat)


# speedup vs baseline: 1.0141x; 1.0141x over previous
import jax
import jax.numpy as jnp
from jax.experimental import pallas as pl
from jax.experimental.pallas import tpu as pltpu


def _bisect_steps(a, k, lo, hi, steps):
    for _ in range(steps):
        mid = jnp.floor((lo + hi + 1.0) * 0.5)
        cnt = jnp.sum((a >= mid).astype(jnp.float32), axis=-1, keepdims=True)
        ge = cnt >= k
        lo = jnp.where(ge, mid, lo)
        hi = jnp.where(ge, hi, mid - 1.0)
    return lo, hi


def _bisect_finish(a, k, lo, hi):
    def cond(carry):
        lo, hi = carry
        return jnp.max(hi - lo) > 0.0

    def body(carry):
        return _bisect_steps(a, k, *carry, steps=2)

    lo, _ = jax.lax.while_loop(cond, body, (lo, hi))
    return lo


def _row_min_max(a):
    n = a.shape[-1]
    mn = a[:, :128]
    mx = mn
    for c in range(128, n, 128):
        blk = a[:, c : c + 128]
        mn = jnp.minimum(mn, blk)
        mx = jnp.maximum(mx, blk)
    return (
        jnp.min(mn, axis=-1, keepdims=True),
        jnp.max(mx, axis=-1, keepdims=True),
    )


def _wta_block(x_ref, wxh_ref, wcat_ref, y_ref, cat_ref):
    x = x_ref[...].astype(jnp.bfloat16)
    a_h = jnp.dot(x, wxh_ref[...], preferred_element_type=jnp.float32)
    hi_h = jnp.max(a_h, axis=-1, keepdims=True)
    lo_h, hi_h = _bisect_steps(a_h, 13, jnp.zeros_like(hi_h), hi_h, steps=4)
    thr_h = _bisect_finish(a_h, 13, lo_h, hi_h)
    h = jnp.where(a_h >= thr_h, a_h, 0.0).astype(jnp.bfloat16)
    cat_ref[:, : x.shape[1]] = x
    cat_ref[:, x.shape[1] :] = h
    a_y = jnp.dot(cat_ref[...], wcat_ref[...], preferred_element_type=jnp.float32)
    lo_y, hi_y = _row_min_max(a_y)
    lo_y, hi_y = _bisect_steps(a_y, 51, lo_y, hi_y, steps=6)
    thr_y = _bisect_finish(a_y, 51, lo_y, hi_y)
    y_ref[...] = jnp.where(a_y >= thr_y, a_y, 0.0)


@jax.jit
def _wta(x, w_xh, w_cat):
    B, NX = x.shape
    NH = w_xh.shape[1]
    NY = w_cat.shape[1]
    BLK = 1024
    grid = (B // BLK,)
    return pl.pallas_call(
        _wta_block,
        grid=grid,
        in_specs=[
            pl.BlockSpec((BLK, NX), lambda i: (i, 0)),
            pl.BlockSpec((NX, NH), lambda i: (0, 0)),
            pl.BlockSpec((NX + NH, NY), lambda i: (0, 0)),
        ],
        out_specs=pl.BlockSpec((BLK, NY), lambda i: (i, 0)),
        out_shape=jax.ShapeDtypeStruct((B, NY), jnp.float32),
        scratch_shapes=[pltpu.VMEM((BLK, NX + NH), jnp.bfloat16)],
    )(x, w_xh, w_cat)


def kernel(x, w_xy, w_xh, w_hy, k_y, k_h):
    w_cat = jnp.concatenate(
        [w_xy.astype(jnp.bfloat16), -w_hy.astype(jnp.bfloat16)], axis=0
    )
    return _wta(x, w_xh.astype(jnp.bfloat16), w_cat)


# speedup vs baseline: 1.0160x; 1.0019x over previous
import jax
import jax.numpy as jnp
from jax.experimental import pallas as pl
from jax.experimental.pallas import tpu as pltpu


def _bisect_steps(a, k, lo, hi, steps):
    for _ in range(steps):
        mid = jnp.floor((lo + hi + 1.0) * 0.5)
        cnt = jnp.sum((a >= mid).astype(jnp.float32), axis=-1, keepdims=True)
        ge = cnt >= k
        lo = jnp.where(ge, mid, lo)
        hi = jnp.where(ge, hi, mid - 1.0)
    return lo, hi


def _bisect_finish(a, k, lo, hi):
    def cond(carry):
        lo, hi = carry
        return jnp.max(hi - lo) > 0.0

    def body(carry):
        return _bisect_steps(a, k, *carry, steps=2)

    lo, _ = jax.lax.while_loop(cond, body, (lo, hi))
    return lo


def _row_min_max(a):
    n = a.shape[-1]
    mn = a[:, :128]
    mx = mn
    for c in range(128, n, 128):
        blk = a[:, c : c + 128]
        mn = jnp.minimum(mn, blk)
        mx = jnp.maximum(mx, blk)
    return (
        jnp.min(mn, axis=-1, keepdims=True),
        jnp.max(mx, axis=-1, keepdims=True),
    )


def _wta_block(x_ref, wxh_ref, wcat_ref, y_ref, cat_ref):
    x = x_ref[...].astype(jnp.bfloat16)
    a_h = jnp.dot(x, wxh_ref[...], preferred_element_type=jnp.float32)
    half = a_h.shape[0] // 2
    ah = (a_h[:half], a_h[half:])
    hih = [jnp.max(a, axis=-1, keepdims=True) for a in ah]
    bh = [
        _bisect_steps(a, 13, jnp.zeros_like(hi), hi, steps=4)
        for a, hi in zip(ah, hih)
    ]
    thr_h = [_bisect_finish(a, 13, lo, hi) for a, (lo, hi) in zip(ah, bh)]
    cat_ref[:, : x.shape[1]] = x
    for p in range(2):
        cat_ref[p * half : (p + 1) * half, x.shape[1] :] = jnp.where(
            ah[p] >= thr_h[p], ah[p], 0.0
        ).astype(jnp.bfloat16)
    a_y = jnp.dot(cat_ref[...], wcat_ref[...], preferred_element_type=jnp.float32)
    ay = (a_y[:half], a_y[half:])
    mm = [_row_min_max(a) for a in ay]
    by = [_bisect_steps(a, 51, lo, hi, steps=6) for a, (lo, hi) in zip(ay, mm)]
    thr_y = [_bisect_finish(a, 51, lo, hi) for a, (lo, hi) in zip(ay, by)]
    for p in range(2):
        y_ref[p * half : (p + 1) * half, :] = jnp.where(
            ay[p] >= thr_y[p], ay[p], 0.0
        )


@jax.jit
def _wta(x, w_xh, w_cat):
    B, NX = x.shape
    NH = w_xh.shape[1]
    NY = w_cat.shape[1]
    BLK = 1024
    grid = (B // BLK,)
    return pl.pallas_call(
        _wta_block,
        grid=grid,
        in_specs=[
            pl.BlockSpec((BLK, NX), lambda i: (i, 0)),
            pl.BlockSpec((NX, NH), lambda i: (0, 0)),
            pl.BlockSpec((NX + NH, NY), lambda i: (0, 0)),
        ],
        out_specs=pl.BlockSpec((BLK, NY), lambda i: (i, 0)),
        out_shape=jax.ShapeDtypeStruct((B, NY), jnp.float32),
        scratch_shapes=[pltpu.VMEM((BLK, NX + NH), jnp.bfloat16)],
    )(x, w_xh, w_cat)


def kernel(x, w_xy, w_xh, w_hy, k_y, k_h):
    w_cat = jnp.concatenate(
        [w_xy.astype(jnp.bfloat16), -w_hy.astype(jnp.bfloat16)], axis=0
    )
    return _wta(x, w_xh.astype(jnp.bfloat16), w_cat)


# speedup vs baseline: 1.0459x; 1.0294x over previous
import jax
import jax.numpy as jnp
from jax.experimental import pallas as pl
from jax.experimental.pallas import tpu as pltpu


def _bisect_steps(a, k, lo, hi, steps):
    for _ in range(steps):
        mid = jnp.floor((lo + hi + 1.0) * 0.5)
        cnt = jnp.sum((a >= mid).astype(jnp.float32), axis=-1, keepdims=True)
        ge = cnt >= k
        lo = jnp.where(ge, mid, lo)
        hi = jnp.where(ge, hi, mid - 1.0)
    return lo, hi


def _bisect_finish(a, k, lo, hi):
    def cond(carry):
        lo, hi = carry
        return jnp.max(hi - lo) > 0.0

    def body(carry):
        return _bisect_steps(a, k, *carry, steps=2)

    lo, _ = jax.lax.while_loop(cond, body, (lo, hi))
    return lo


def _row_min_max(a):
    n = a.shape[-1]
    mn = a[:, :128]
    mx = mn
    for c in range(128, n, 128):
        blk = a[:, c : c + 128]
        mn = jnp.minimum(mn, blk)
        mx = jnp.maximum(mx, blk)
    return (
        jnp.min(mn, axis=-1, keepdims=True),
        jnp.max(mx, axis=-1, keepdims=True),
    )


def _wta_block(x_ref, wxh_ref, wcat_ref, y_ref, cat_ref):
    NX = x_ref.shape[1]
    half = x_ref.shape[0] // 2
    rows = (slice(0, half), slice(half, 2 * half))


    x = x_ref[...].astype(jnp.bfloat16)
    cat_ref[:, :NX] = x
    a_h = [
        jnp.dot(x[r], wxh_ref[...], preferred_element_type=jnp.float32)
        for r in rows
    ]
    hi_h = [jnp.max(a, axis=-1, keepdims=True) for a in a_h]
    pre_h = [
        _bisect_steps(a, 13, jnp.zeros_like(hi), hi, steps=4)
        for a, hi in zip(a_h, hi_h)
    ]
    thr_h = [_bisect_finish(a, 13, lo, hi) for a, (lo, hi) in zip(a_h, pre_h)]

    for p, r in enumerate(rows):
        cat_ref[r, NX:] = jnp.where(a_h[p] >= thr_h[p], a_h[p], 0.0).astype(
            jnp.bfloat16
        )
    a_y0 = jnp.dot(
        cat_ref[rows[0], :], wcat_ref[...], preferred_element_type=jnp.float32
    )
    a_y1 = jnp.dot(
        cat_ref[rows[1], :], wcat_ref[...], preferred_element_type=jnp.float32
    )
    lo0, hi0 = _row_min_max(a_y0)
    lo0, hi0 = _bisect_steps(a_y0, 51, lo0, hi0, steps=6)

    thr0 = _bisect_finish(a_y0, 51, lo0, hi0)

    y_ref[rows[0], :] = jnp.where(a_y0 >= thr0, a_y0, 0.0)
    lo1, hi1 = _row_min_max(a_y1)
    lo1, hi1 = _bisect_steps(a_y1, 51, lo1, hi1, steps=6)

    thr1 = _bisect_finish(a_y1, 51, lo1, hi1)
    y_ref[rows[1], :] = jnp.where(a_y1 >= thr1, a_y1, 0.0)


@jax.jit
def _wta(x, w_xh, w_cat):
    B, NX = x.shape
    NH = w_xh.shape[1]
    NY = w_cat.shape[1]
    BLK = 1024
    grid = (B // BLK,)
    return pl.pallas_call(
        _wta_block,
        grid=grid,
        in_specs=[
            pl.BlockSpec((BLK, NX), lambda i: (i, 0)),
            pl.BlockSpec((NX, NH), lambda i: (0, 0)),
            pl.BlockSpec((NX + NH, NY), lambda i: (0, 0)),
        ],
        out_specs=pl.BlockSpec((BLK, NY), lambda i: (i, 0)),
        out_shape=jax.ShapeDtypeStruct((B, NY), jnp.float32),
        scratch_shapes=[pltpu.VMEM((BLK, NX + NH), jnp.bfloat16)],
    )(x, w_xh, w_cat)


def kernel(x, w_xy, w_xh, w_hy, k_y, k_h):
    w_cat = jnp.concatenate(
        [w_xy.astype(jnp.bfloat16), -w_hy.astype(jnp.bfloat16)], axis=0
    )
    return _wta(x, w_xh.astype(jnp.bfloat16), w_cat)


# speedup vs baseline: 1.0460x; 1.0001x over previous
import jax
import jax.numpy as jnp
from jax.experimental import pallas as pl
from jax.experimental.pallas import tpu as pltpu


def _bisect_steps(a, k, lo, hi, steps):
    for _ in range(steps):
        mid = jnp.floor((lo + hi + 1.0) * 0.5)
        cnt = jnp.sum((a >= mid).astype(jnp.float32), axis=-1, keepdims=True)
        ge = cnt >= k
        lo = jnp.where(ge, mid, lo)
        hi = jnp.where(ge, hi, mid - 1.0)
    return lo, hi


def _bisect_finish(a, k, lo, hi):
    def cond(carry):
        lo, hi = carry
        return jnp.max(hi - lo) > 0.0

    def body(carry):
        return _bisect_steps(a, k, *carry, steps=2)

    lo, _ = jax.lax.while_loop(cond, body, (lo, hi))
    return lo


def _row_min_max(a):
    n = a.shape[-1]
    mn = a[:, :128]
    mx = mn
    for c in range(128, n, 128):
        blk = a[:, c : c + 128]
        mn = jnp.minimum(mn, blk)
        mx = jnp.maximum(mx, blk)
    return (
        jnp.min(mn, axis=-1, keepdims=True),
        jnp.max(mx, axis=-1, keepdims=True),
    )


def _wta_block(x_ref, wxh_ref, wcat_ref, y_ref, cat_ref):
    NX = x_ref.shape[1]
    half = x_ref.shape[0] // 2
    rows = (slice(0, half), slice(half, 2 * half))


    x = x_ref[...].astype(jnp.bfloat16)
    cat_ref[:, :NX] = x
    a_h = [
        jnp.dot(x[r], wxh_ref[...], preferred_element_type=jnp.float32)
        for r in rows
    ]
    hi_h = [jnp.max(a, axis=-1, keepdims=True) for a in a_h]
    pre_h = [
        _bisect_steps(a, 13, jnp.zeros_like(hi), hi, steps=4)
        for a, hi in zip(a_h, hi_h)
    ]
    thr_h = [_bisect_finish(a, 13, lo, hi) for a, (lo, hi) in zip(a_h, pre_h)]

    for p, r in enumerate(rows):
        cat_ref[r, NX:] = jnp.where(a_h[p] >= thr_h[p], a_h[p], 0.0).astype(
            jnp.bfloat16
        )
    a_y0 = jnp.dot(
        cat_ref[rows[0], :], wcat_ref[...], preferred_element_type=jnp.float32
    )
    lo0, hi0 = _row_min_max(a_y0)
    lo0, hi0 = _bisect_steps(a_y0, 51, lo0, hi0, steps=6)
    a_y1 = jnp.dot(
        cat_ref[rows[1], :], wcat_ref[...], preferred_element_type=jnp.float32
    )

    thr0 = _bisect_finish(a_y0, 51, lo0, hi0)

    y_ref[rows[0], :] = jnp.where(a_y0 >= thr0, a_y0, 0.0)
    lo1, hi1 = _row_min_max(a_y1)
    lo1, hi1 = _bisect_steps(a_y1, 51, lo1, hi1, steps=6)

    thr1 = _bisect_finish(a_y1, 51, lo1, hi1)
    y_ref[rows[1], :] = jnp.where(a_y1 >= thr1, a_y1, 0.0)


@jax.jit
def _wta(x, w_xh, w_cat):
    B, NX = x.shape
    NH = w_xh.shape[1]
    NY = w_cat.shape[1]
    BLK = 1024
    grid = (B // BLK,)
    return pl.pallas_call(
        _wta_block,
        grid=grid,
        in_specs=[
            pl.BlockSpec((BLK, NX), lambda i: (i, 0)),
            pl.BlockSpec((NX, NH), lambda i: (0, 0)),
            pl.BlockSpec((NX + NH, NY), lambda i: (0, 0)),
        ],
        out_specs=pl.BlockSpec((BLK, NY), lambda i: (i, 0)),
        out_shape=jax.ShapeDtypeStruct((B, NY), jnp.float32),
        scratch_shapes=[pltpu.VMEM((BLK, NX + NH), jnp.bfloat16)],
    )(x, w_xh, w_cat)


def kernel(x, w_xy, w_xh, w_hy, k_y, k_h):
    w_cat = jnp.concatenate(
        [w_xy.astype(jnp.bfloat16), -w_hy.astype(jnp.bfloat16)], axis=0
    )
    return _wta(x, w_xh.astype(jnp.bfloat16), w_cat)
